# baseline (speedup 1.0000x reference)
import functools

import jax
import jax.numpy as jnp
from jax import lax
from jax.experimental import pallas as pl
from jax.experimental.pallas import tpu as pltpu
from jax.experimental.pallas import tpu_sc as plsc

_INFO = plsc.get_sparse_core_info()
_NC, _NS = _INFO.num_cores, _INFO.num_subcores
_NW = _NC * _NS

_R, _C, _D = 16384, 200, 6
_V = 576
_B = _R * _C
_PER_TILE = _B // _NW
_CHUNK = 1600
_NCH = _PER_TILE // _CHUNK
_NG = _CHUNK // 16
_OW = _CHUNK * _D


def _body(state_hbm, poly_hbm, out_hbm, pt_v, tab_v, idx_v, rows_v, s_out0, s_out1):
    wid = lax.axis_index("s") * _NC + lax.axis_index("c")
    i_base = wid * _PER_TILE

    pltpu.sync_copy(poly_hbm, pt_v)
    iota16 = lax.iota(jnp.int32, 16)
    colid = [jnp.full((16,), j, jnp.int32) for j in range(_D)]
    for g in range(_V // 16):
        sl = pl.ds(g * 16, 16)
        c = [pt_v[j, sl] for j in range(_D)]
        m = c[0]
        for j in range(1, _D):
            m = jnp.maximum(m, c[j])
        e = [jnp.exp(c[j] - m) for j in range(_D)]
        s = e[0]
        for j in range(1, _D):
            s = s + e[j]
        inv = 1.0 / s
        rows16 = iota16 + (g * 16)
        for j in range(_D):
            plsc.store_scatter(tab_v, [rows16, colid[j]], e[j] * inv)

    sems = (s_out0, s_out1)
    iota6 = iota16 * _D

    def pair(it, carry):
        for b in (0, 1):
            ch = it * 2 + b
            w_base = (i_base + ch * _CHUNK) * _D

            @pl.when(it >= 1)
            def _wait_prev():
                pltpu.make_async_copy(
                    rows_v.at[b], out_hbm.at[pl.ds(w_base, _OW)], sems[b]
                ).wait()

            pltpu.sync_copy(
                state_hbm.at[pl.ds(i_base + ch * _CHUNK, _CHUNK)], idx_v
            )
            for g in range(_NG):
                ii = idx_v[pl.ds(g * 16, 16)]
                fl = iota6 + (g * 16 * _D)
                for j in range(_D):
                    v = plsc.load_gather(tab_v, [ii, colid[j]])
                    plsc.store_scatter(rows_v.at[b], [fl + j], v)
            pltpu.async_copy(
                rows_v.at[b], out_hbm.at[pl.ds(w_base, _OW)], sems[b]
            )
        return carry

    lax.fori_loop(0, _NCH // 2, pair, 0)

    for b in (0, 1):
        w_last = (i_base + (_NCH - 2 + b) * _CHUNK) * _D
        pltpu.make_async_copy(
            rows_v.at[b], out_hbm.at[pl.ds(w_last, _OW)], sems[b]
        ).wait()


def _sc_lookup(state_flat, policy_t):
    mesh = plsc.VectorSubcoreMesh(core_axis_name="c", subcore_axis_name="s")
    kern = functools.partial(
        pl.kernel,
        mesh=mesh,
        out_type=jax.ShapeDtypeStruct((_B * _D,), jnp.float32),
        scratch_types=[
            pltpu.VMEM((_D, _V), jnp.float32),
            pltpu.VMEM((_V, _D), jnp.float32),
            pltpu.VMEM((_CHUNK,), jnp.int32),
            pltpu.VMEM((2, _OW), jnp.float32),
            pltpu.SemaphoreType.DMA,
            pltpu.SemaphoreType.DMA,
        ],
        compiler_params=pltpu.CompilerParams(
            use_tc_tiling_on_sc=False, needs_layout_passes=False
        ),
    )(_body)
    return kern(state_flat, policy_t)


def kernel(state, policy):
    state_flat = state.astype(jnp.int32).reshape(_B)
    policy_t = policy.astype(jnp.float32).T.reshape(_D, _V)
    out = _sc_lookup(state_flat, policy_t)
    return out.reshape(_R, _C, _D)


# speedup vs baseline: 9.0962x; 9.0962x over previous
import functools

import jax
import jax.numpy as jnp
from jax import lax
from jax.experimental import pallas as pl
from jax.experimental.pallas import tpu as pltpu
from jax.experimental.pallas import tpu_sc as plsc

_INFO = plsc.get_sparse_core_info()
_NC, _NS = _INFO.num_cores, _INFO.num_subcores
_NW = _NC * _NS

_R, _C, _D = 16384, 200, 6
_V = 576
_B = _R * _C
_A = _C // 8
_RT = _R // _NW
_NB = _RT // 128
_CW = _NB * 8 * 128
_NGK = _RT // 16


def _body(state_hbm, poly_hbm, out_hbm, pt_v, tab_v, idx_v, rows_v,
          s_idx0, s_idx1, s_out0, s_out1):
    wid = lax.axis_index("s") * _NC + lax.axis_index("c")
    r0 = wid * _RT

    pltpu.sync_copy(poly_hbm, pt_v)
    iota16 = lax.iota(jnp.int32, 16)
    colid = [jnp.full((16,), j, jnp.int32) for j in range(_D)]
    for g in range(_V // 16):
        sl = pl.ds(g * 16, 16)
        c = [pt_v[j, sl] for j in range(_D)]
        m = c[0]
        for j in range(1, _D):
            m = jnp.maximum(m, c[j])
        e = [jnp.exp(c[j] - m) for j in range(_D)]
        s = e[0]
        for j in range(1, _D):
            s = s + e[j]
        inv = 1.0 / s
        rows16 = iota16 + (g * 16)
        for j in range(_D):
            plsc.store_scatter(tab_v, [rows16, colid[j]], e[j] * inv)

    s_idx = (s_idx0, s_idx1)
    s_out = (s_out0, s_out1)

    def fetch_idx(a, bb):
        for i in range(8):
            pltpu.async_copy(
                state_hbm.at[pl.ds((a * 8 + i) * _R + r0, _RT)],
                idx_v.at[bb, i],
                s_idx[bb],
            )

    def drain_idx(bb):
        for i in range(8):
            pltpu.make_async_copy(
                state_hbm.at[pl.ds(r0, _RT)], idx_v.at[bb, i], s_idx[bb]
            ).wait()

    def gather_chunk(bb):
        def g_body(g, carry):
            goff = g * 16
            off0 = ((g >> 3) << 10) + ((g & 7) << 4)
            for i in range(8):
                ii = idx_v[bb, i, pl.ds(goff, 16)]
                off = off0 + i * 128
                for d in range(_D):
                    v = plsc.load_gather(tab_v, [ii, colid[d]])
                    rows_v[bb, d, pl.ds(off, 16)] = v
            return carry

        lax.fori_loop(0, _NGK, g_body, 0)

    def out_dma(a, bb, wait_only):
        for d in range(_D):
            off = (d * _A + a) * (_NW * _CW) + wid * _CW
            cp = pltpu.make_async_copy(
                rows_v.at[bb, d], out_hbm.at[pl.ds(off, _CW)], s_out[bb]
            )
            if wait_only:
                cp.wait()
            else:
                cp.start()

    fetch_idx(0, 0)

    def pair(t, carry):
        for bb in (0, 1):
            a = t * 2 + bb

            @pl.when(t >= 1)
            def _wait_out():
                out_dma(a, bb, True)

            drain_idx(bb)

            @pl.when(a + 1 < _A)
            def _prefetch():
                fetch_idx(a + 1, 1 - bb)

            gather_chunk(bb)
            out_dma(a, bb, False)
        return carry

    lax.fori_loop(0, _A // 2, pair, 0)

    a_t = _A - 1
    out_dma(a_t, 0, True)
    drain_idx(0)
    gather_chunk(0)
    out_dma(a_t, 0, False)
    out_dma(a_t - 1, 1, True)
    out_dma(a_t, 0, True)


def _sc_lookup(state_t_flat, policy_t):
    mesh = plsc.VectorSubcoreMesh(core_axis_name="c", subcore_axis_name="s")
    kern = functools.partial(
        pl.kernel,
        mesh=mesh,
        out_type=jax.ShapeDtypeStruct((_B * _D,), jnp.float32),
        scratch_types=[
            pltpu.VMEM((_D, _V), jnp.float32),
            pltpu.VMEM((_V, _D), jnp.float32),
            pltpu.VMEM((2, 8, _RT), jnp.int32),
            pltpu.VMEM((2, _D, _CW), jnp.float32),
            pltpu.SemaphoreType.DMA,
            pltpu.SemaphoreType.DMA,
            pltpu.SemaphoreType.DMA,
            pltpu.SemaphoreType.DMA,
        ],
        compiler_params=pltpu.CompilerParams(
            use_tc_tiling_on_sc=False, needs_layout_passes=False
        ),
    )(_body)
    return kern(state_t_flat, policy_t)


def kernel(state, policy):
    state_t_flat = state.astype(jnp.int32).T.reshape(_B)
    policy_t = policy.astype(jnp.float32).T.reshape(_D, _V)
    out = _sc_lookup(state_t_flat, policy_t)
    x5 = out.reshape(_D, _A, _R // 128, 8, 128)
    xt = x5.transpose(2, 4, 1, 3, 0)
    return xt.reshape(_R, _C, _D)


# speedup vs baseline: 29.0727x; 3.1961x over previous
import functools

import jax
import jax.numpy as jnp
from jax import lax
from jax.experimental import pallas as pl
from jax.experimental.pallas import tpu as pltpu
from jax.experimental.pallas import tpu_sc as plsc

_INFO = plsc.get_sparse_core_info()
_NC, _NS = _INFO.num_cores, _INFO.num_subcores
_NW = _NC * _NS

_R, _C, _D = 16384, 200, 6
_V = 576
_B = _R * _C
_A = _C // 8
_NQ = 16
_RT = _R // _NQ
_CW = _RT * 8
_NGK = _RT // 16
_NSLOT = (_A + 1) // 2


def _body(state_hbm, poly_hbm, out_hbm, pt_v, tab_v, idx_v, rows_v,
          s_idx0, s_idx1, s_out0, s_out1):
    wid = lax.axis_index("s") * _NC + lax.axis_index("c")
    q = wid // 2
    p = wid - q * 2
    r0 = q * _RT

    pltpu.sync_copy(poly_hbm, pt_v)
    iota16 = lax.iota(jnp.int32, 16)
    colid = [jnp.full((16,), j, jnp.int32) for j in range(_D)]
    for g in range(_V // 16):
        sl = pl.ds(g * 16, 16)
        c = [pt_v[j, sl] for j in range(_D)]
        m = c[0]
        for j in range(1, _D):
            m = jnp.maximum(m, c[j])
        e = [jnp.exp(c[j] - m) for j in range(_D)]
        s = e[0]
        for j in range(1, _D):
            s = s + e[j]
        inv = 1.0 / s
        rows16 = iota16 + (g * 16)
        for j in range(_D):
            plsc.store_scatter(tab_v, [rows16, colid[j]], e[j] * inv)

    s_idx = (s_idx0, s_idx1)
    s_out = (s_out0, s_out1)

    def fetch_idx(a, bb):
        for i in range(8):
            pltpu.async_copy(
                state_hbm.at[pl.ds((a * 8 + i) * _R + r0, _RT)],
                idx_v.at[bb, i],
                s_idx[bb],
            )

    def drain_idx(bb):
        for i in range(8):
            pltpu.make_async_copy(
                state_hbm.at[pl.ds(r0, _RT)], idx_v.at[bb, i], s_idx[bb]
            ).wait()

    def gather_chunk(bb):
        @plsc.parallel_loop(0, _NGK, unroll=2)
        def g_body(g):
            goff = g * 16
            off0 = ((g >> 3) << 10) + ((g & 7) << 4)
            for i in range(8):
                ii = idx_v[bb, i, pl.ds(goff, 16)]
                off = off0 + i * 128
                vals = [plsc.load_gather(tab_v, [ii, colid[d]]) for d in range(_D)]
                for d in range(_D):
                    rows_v[bb, d, pl.ds(off, 16)] = vals[d]

    def out_dma(a, bb, wait_only):
        for d in range(_D):
            off = (d * _A + a) * (_NQ * _CW) + q * _CW
            cp = pltpu.make_async_copy(
                rows_v.at[bb, d], out_hbm.at[pl.ds(off, _CW)], s_out[bb]
            )
            if wait_only:
                cp.wait()
            else:
                cp.start()

    fetch_idx(p, 0)

    def pair(t, carry):
        for bb in (0, 1):
            n = t * 2 + bb
            a = p + 2 * n

            @pl.when(t >= 1)
            def _wait_out():
                out_dma(a, bb, True)

            drain_idx(bb)
            a_next = a + 2

            @pl.when(a_next < _A)
            def _prefetch():
                fetch_idx(a_next, 1 - bb)

            gather_chunk(bb)
            out_dma(a, bb, False)
        return carry

    lax.fori_loop(0, (_NSLOT - 1) // 2, pair, 0)

    a_t = _A - 1

    @pl.when(p == 0)
    def _tail():
        out_dma(a_t, 0, True)
        drain_idx(0)
        gather_chunk(0)
        out_dma(a_t, 0, False)

    out_dma(0, 0, True)
    out_dma(0, 1, True)


def _sc_lookup(state_t_flat, policy_t):
    mesh = plsc.VectorSubcoreMesh(core_axis_name="c", subcore_axis_name="s")
    kern = functools.partial(
        pl.kernel,
        mesh=mesh,
        out_type=jax.ShapeDtypeStruct((_B * _D,), jnp.float32),
        scratch_types=[
            pltpu.VMEM((_D, _V), jnp.float32),
            pltpu.VMEM((_V, _D), jnp.float32),
            pltpu.VMEM((2, 8, _RT), jnp.int32),
            pltpu.VMEM((2, _D, _CW), jnp.float32),
            pltpu.SemaphoreType.DMA,
            pltpu.SemaphoreType.DMA,
            pltpu.SemaphoreType.DMA,
            pltpu.SemaphoreType.DMA,
        ],
        compiler_params=pltpu.CompilerParams(
            use_tc_tiling_on_sc=False, needs_layout_passes=False
        ),
    )(_body)
    return kern(state_t_flat, policy_t)


def kernel(state, policy):
    state_t_flat = state.astype(jnp.int32).T.reshape(_B)
    policy_t = policy.astype(jnp.float32).T.reshape(_D, _V)
    out = _sc_lookup(state_t_flat, policy_t)
    x5 = out.reshape(_D, _A, _R // 128, 8, 128)
    xt = x5.transpose(2, 4, 1, 3, 0)
    return xt.reshape(_R, _C, _D)


# speedup vs baseline: 29.4754x; 1.0139x over previous
import functools

import jax
import jax.numpy as jnp
from jax import lax
from jax.experimental import pallas as pl
from jax.experimental.pallas import tpu as pltpu
from jax.experimental.pallas import tpu_sc as plsc

_INFO = plsc.get_sparse_core_info()
_NC, _NS = _INFO.num_cores, _INFO.num_subcores
_NW = _NC * _NS

_R, _C, _D = 16384, 200, 6
_V = 576
_B = _R * _C
_A = _C // 8
_RT = _R // _NW
_NB = _RT // 128
_CW = _NB * 8 * 128
_NGK = _RT // 16


def _body(state_hbm, poly_hbm, out_hbm, pt_v, tab_v, idx_v, rows_v,
          s_idx0, s_idx1, s_out0, s_out1):
    wid = lax.axis_index("s") * _NC + lax.axis_index("c")
    r0 = wid * _RT

    pltpu.sync_copy(poly_hbm, pt_v)
    iota16 = lax.iota(jnp.int32, 16)
    colid = [jnp.full((16,), j, jnp.int32) for j in range(_D)]
    for g in range(_V // 16):
        sl = pl.ds(g * 16, 16)
        c = [pt_v[j, sl] for j in range(_D)]
        m = c[0]
        for j in range(1, _D):
            m = jnp.maximum(m, c[j])
        e = [jnp.exp(c[j] - m) for j in range(_D)]
        s = e[0]
        for j in range(1, _D):
            s = s + e[j]
        inv = 1.0 / s
        rows16 = iota16 + (g * 16)
        for j in range(_D):
            plsc.store_scatter(tab_v, [rows16, colid[j]], e[j] * inv)

    s_idx = (s_idx0, s_idx1)
    s_out = (s_out0, s_out1)

    def fetch_idx(a, bb):
        for i in range(8):
            pltpu.async_copy(
                state_hbm.at[pl.ds((a * 8 + i) * _R + r0, _RT)],
                idx_v.at[bb, i],
                s_idx[bb],
            )

    def drain_idx(bb):
        for i in range(8):
            pltpu.make_async_copy(
                state_hbm.at[pl.ds(r0, _RT)], idx_v.at[bb, i], s_idx[bb]
            ).wait()

    def gather_chunk(bb):
        @plsc.parallel_loop(0, _NGK, unroll=2)
        def g_body(g):
            goff = g * 16
            off0 = ((g >> 3) << 10) + ((g & 7) << 4)
            for i in range(8):
                ii = idx_v[bb, i, pl.ds(goff, 16)]
                off = off0 + i * 128
                vals = [plsc.load_gather(tab_v, [ii, colid[d]]) for d in range(_D)]
                for d in range(_D):
                    rows_v[bb, d, pl.ds(off, 16)] = vals[d]

    def out_dma(a, bb, wait_only):
        for d in range(_D):
            off = (d * _A + a) * (_NW * _CW) + wid * _CW
            cp = pltpu.make_async_copy(
                rows_v.at[bb, d], out_hbm.at[pl.ds(off, _CW)], s_out[bb]
            )
            if wait_only:
                cp.wait()
            else:
                cp.start()

    fetch_idx(0, 0)

    def pair(t, carry):
        for bb in (0, 1):
            a = t * 2 + bb

            @pl.when(t >= 1)
            def _wait_out():
                out_dma(a, bb, True)

            drain_idx(bb)

            @pl.when(a + 1 < _A)
            def _prefetch():
                fetch_idx(a + 1, 1 - bb)

            gather_chunk(bb)
            out_dma(a, bb, False)
        return carry

    lax.fori_loop(0, _A // 2, pair, 0)

    a_t = _A - 1
    out_dma(a_t, 0, True)
    drain_idx(0)
    gather_chunk(0)
    out_dma(a_t, 0, False)
    out_dma(a_t - 1, 1, True)
    out_dma(a_t, 0, True)


def _sc_lookup(state_t_flat, policy_t):
    mesh = plsc.VectorSubcoreMesh(core_axis_name="c", subcore_axis_name="s")
    kern = functools.partial(
        pl.kernel,
        mesh=mesh,
        out_type=jax.ShapeDtypeStruct((_B * _D,), jnp.float32),
        scratch_types=[
            pltpu.VMEM((_D, _V), jnp.float32),
            pltpu.VMEM((_V, _D), jnp.float32),
            pltpu.VMEM((2, 8, _RT), jnp.int32),
            pltpu.VMEM((2, _D, _CW), jnp.float32),
            pltpu.SemaphoreType.DMA,
            pltpu.SemaphoreType.DMA,
            pltpu.SemaphoreType.DMA,
            pltpu.SemaphoreType.DMA,
        ],
        compiler_params=pltpu.CompilerParams(
            use_tc_tiling_on_sc=False, needs_layout_passes=False
        ),
    )(_body)
    return kern(state_t_flat, policy_t)


def kernel(state, policy):
    state_t_flat = state.astype(jnp.int32).T.reshape(_B)
    policy_t = policy.astype(jnp.float32).T.reshape(_D, _V)
    out = _sc_lookup(state_t_flat, policy_t)
    x5 = out.reshape(_D, _A, _R // 128, 8, 128)
    xt = x5.transpose(2, 4, 1, 3, 0)
    return xt.reshape(_R, _C, _D)


# speedup vs baseline: 33.3267x; 1.1307x over previous
import functools

import jax
import jax.numpy as jnp
from jax import lax
from jax.experimental import pallas as pl
from jax.experimental.pallas import tpu as pltpu
from jax.experimental.pallas import tpu_sc as plsc

_INFO = plsc.get_sparse_core_info()
_NC, _NS = _INFO.num_cores, _INFO.num_subcores
_NW = _NC * _NS

_R, _C, _D = 16384, 200, 6
_V = 576
_B = _R * _C
_A = _C // 8
_RT = _R // _NW
_NB = _RT // 128
_CW = _NB * 8 * 128
_NGK = _RT // 16


def _body(state_hbm, poly_hbm, out_hbm, pt_v, tab_v, idx_v, rows_v,
          s_idx0, s_idx1, s_out0, s_out1):
    wid = lax.axis_index("s") * _NC + lax.axis_index("c")
    r0 = wid * _RT

    pltpu.sync_copy(poly_hbm, pt_v)
    iota16 = lax.iota(jnp.int32, 16)
    colid = [jnp.full((16,), j, jnp.int32) for j in range(_D)]
    for g in range(_V // 16):
        sl = pl.ds(g * 16, 16)
        c = [pt_v[j, sl] for j in range(_D)]
        m = c[0]
        for j in range(1, _D):
            m = jnp.maximum(m, c[j])
        e = [jnp.exp(c[j] - m) for j in range(_D)]
        s = e[0]
        for j in range(1, _D):
            s = s + e[j]
        inv = 1.0 / s
        rows16 = iota16 + (g * 16)
        for j in range(_D):
            plsc.store_scatter(tab_v, [rows16, colid[j]], e[j] * inv)

    s_idx = (s_idx0, s_idx1)
    s_out = (s_out0, s_out1)

    def fetch_idx(a, bb):
        pltpu.async_copy(
            state_hbm.at[pl.ds(a * (_NW * _CW) + wid * _CW, _CW)],
            idx_v.at[bb],
            s_idx[bb],
        )

    def drain_idx(bb):
        pltpu.make_async_copy(
            state_hbm.at[pl.ds(r0, _CW)], idx_v.at[bb], s_idx[bb]
        ).wait()

    def gather_chunk(bb):
        @plsc.parallel_loop(0, _NGK, unroll=2)
        def g_body(g):
            off0 = ((g >> 3) << 10) + ((g & 7) << 4)
            for i in range(8):
                off = off0 + i * 128
                ii = idx_v[bb, pl.ds(off, 16)]
                vals = [plsc.load_gather(tab_v, [ii, colid[d]]) for d in range(_D)]
                for d in range(_D):
                    rows_v[bb, d, pl.ds(off, 16)] = vals[d]

    def out_dma(a, bb, wait_only):
        for d in range(_D):
            off = (d * _A + a) * (_NW * _CW) + wid * _CW
            cp = pltpu.make_async_copy(
                rows_v.at[bb, d], out_hbm.at[pl.ds(off, _CW)], s_out[bb]
            )
            if wait_only:
                cp.wait()
            else:
                cp.start()

    fetch_idx(0, 0)

    def pair(t, carry):
        for bb in (0, 1):
            a = t * 2 + bb

            @pl.when(t >= 1)
            def _wait_out():
                out_dma(a, bb, True)

            drain_idx(bb)

            @pl.when(a + 1 < _A)
            def _prefetch():
                fetch_idx(a + 1, 1 - bb)

            gather_chunk(bb)
            out_dma(a, bb, False)
        return carry

    lax.fori_loop(0, _A // 2, pair, 0)

    a_t = _A - 1
    out_dma(a_t, 0, True)
    drain_idx(0)
    gather_chunk(0)
    out_dma(a_t, 0, False)
    out_dma(a_t - 1, 1, True)
    out_dma(a_t, 0, True)


def _sc_lookup(state_t_flat, policy_t):
    mesh = plsc.VectorSubcoreMesh(core_axis_name="c", subcore_axis_name="s")
    kern = functools.partial(
        pl.kernel,
        mesh=mesh,
        out_type=jax.ShapeDtypeStruct((_B * _D,), jnp.float32),
        scratch_types=[
            pltpu.VMEM((_D, _V), jnp.float32),
            pltpu.VMEM((_V, _D), jnp.float32),
            pltpu.VMEM((2, _CW), jnp.int32),
            pltpu.VMEM((2, _D, _CW), jnp.float32),
            pltpu.SemaphoreType.DMA,
            pltpu.SemaphoreType.DMA,
            pltpu.SemaphoreType.DMA,
            pltpu.SemaphoreType.DMA,
        ],
        compiler_params=pltpu.CompilerParams(
            use_tc_tiling_on_sc=False, needs_layout_passes=False
        ),
    )(_body)
    return kern(state_t_flat, policy_t)


def kernel(state, policy):
    st = state.astype(jnp.int32).T.reshape(_A, 8, _R // 128, 128)
    state_t_flat = st.transpose(0, 2, 1, 3).reshape(_B)
    policy_t = policy.astype(jnp.float32).T.reshape(_D, _V)
    out = _sc_lookup(state_t_flat, policy_t)
    x5 = out.reshape(_D, _A, _R // 128, 8, 128)
    xt = x5.transpose(2, 4, 1, 3, 0)
    return xt.reshape(_R, _C, _D)
